# baseline (speedup 1.0000x reference)
import functools

import jax
import jax.numpy as jnp
from jax import lax
from jax.experimental import pallas as pl
from jax.experimental.pallas import tpu as pltpu
from jax.experimental.pallas import tpu_sc as plsc

E = 2048
HIDDEN = 128
HEADS = 8
HEAD_DIM = HIDDEN // HEADS
NUM_RADIAL = 64
CUTOFF = 10.0
TOP_K = 32
P = 2 * E * TOP_K

BLK = 2048


def _silu(x):
    return x * jax.nn.sigmoid(x)


def _unpack_pair(w):
    wi = jax.lax.bitcast_convert_type(w, jnp.uint32)
    lo = jax.lax.bitcast_convert_type(wi << 16, jnp.float32)
    hi = jax.lax.bitcast_convert_type(wi & jnp.uint32(0xFFFF0000), jnp.float32)
    return lo, hi


def _pair_mlp_body(g1_ref, g2_ref, cd_ref, aqt_ref, akt_ref, art_ref, ad_ref,
                   ab1_ref, aw2t_ref, ab2_ref, aw3t_ref, cent_ref, ab3_ref,
                   out_ref):
    qe, qo = _unpack_pair(g1_ref[:, :HIDDEN // 2])
    ke, ko = _unpack_pair(g2_ref[:, :HIDDEN // 2])
    cd = cd_ref[...]
    d2 = jnp.sum(cd * cd, axis=-1, keepdims=True)
    d = jnp.sqrt(d2 + 1e-12)
    gamma = (NUM_RADIAL / CUTOFF) ** 2
    cent = cent_ref[...]
    rf = jnp.exp(-gamma * (d - cent) ** 2)
    rfc = jnp.dot(rf, art_ref[...], preferred_element_type=jnp.float32)
    ab1 = ab1_ref[...]
    aw2t = aw2t_ref[...]
    ab2 = ab2_ref[...]
    aw3t = aw3t_ref[...]
    ad = ad_ref[...]
    HW = HEAD_DIM // 2
    for h in range(HEADS):
        qp = jnp.concatenate([qe[:, h * HW:(h + 1) * HW],
                              qo[:, h * HW:(h + 1) * HW]], axis=1)
        kp = jnp.concatenate([ke[:, h * HW:(h + 1) * HW],
                              ko[:, h * HW:(h + 1) * HW]], axis=1)
        dp = jnp.sum(qp * kp, axis=-1, keepdims=True)
        pre = (jnp.dot(qp, aqt_ref[...], preferred_element_type=jnp.float32)
               + jnp.dot(kp, akt_ref[...], preferred_element_type=jnp.float32)
               + rfc + dp * ad + ab1)
        h1 = _silu(pre)
        h2 = _silu(jnp.dot(h1, aw2t, preferred_element_type=jnp.float32) + ab2)
        s = jnp.dot(h2, aw3t[:, h:h + 1], preferred_element_type=jnp.float32)
        out_ref[:, h:h + 1] = jnp.exp(s + ab3_ref[0:1, h:h + 1])


def _pair_scores(g1, g2, cd4, aqt, akt, art, ad, ab1, aw2t, ab2, aw3t, cent,
                 ab3):
    nblk = P // BLK
    row = lambda i: (i, 0)
    fixed = lambda i: (0, 0)
    return pl.pallas_call(
        _pair_mlp_body,
        grid=(nblk,),
        in_specs=[
            pl.BlockSpec((BLK, D1), row),
            pl.BlockSpec((BLK, D2), row),
            pl.BlockSpec((BLK, 4), row),
            pl.BlockSpec((HEAD_DIM, HIDDEN), fixed),
            pl.BlockSpec((HEAD_DIM, HIDDEN), fixed),
            pl.BlockSpec((NUM_RADIAL, HIDDEN), fixed),
            pl.BlockSpec((1, HIDDEN), fixed),
            pl.BlockSpec((1, HIDDEN), fixed),
            pl.BlockSpec((HIDDEN, HIDDEN), fixed),
            pl.BlockSpec((1, HIDDEN), fixed),
            pl.BlockSpec((HIDDEN, HEADS), fixed),
            pl.BlockSpec((1, NUM_RADIAL), fixed),
            pl.BlockSpec((1, HEADS), fixed),
        ],
        out_specs=pl.BlockSpec((BLK, HEADS), row),
        out_shape=jax.ShapeDtypeStruct((P, HEADS), jnp.float32),
    )(g1, g2, cd4, aqt, akt, art, ad, ab1, aw2t, ab2, aw3t, cent, ab3)


D1 = 128
D2 = 128
NW = 32
CHUNK = 128
PER_W = P // NW
NCHUNK = PER_W // CHUNK


def _sc_gather(t1, t2, p0, p1):
    mesh = plsc.VectorSubcoreMesh(core_axis_name="c", subcore_axis_name="s")

    @functools.partial(
        pl.kernel,
        out_type=(jax.ShapeDtypeStruct((P, D1), jnp.float32),
                  jax.ShapeDtypeStruct((P, D2), jnp.float32)),
        mesh=mesh,
        scratch_types=[
            pltpu.VMEM((PER_W,), jnp.int32),
            pltpu.VMEM((PER_W,), jnp.int32),
            pltpu.VMEM((2, CHUNK, D1), jnp.float32),
            pltpu.VMEM((2, CHUNK, D2), jnp.float32),
            pltpu.SemaphoreType.DMA,
            pltpu.SemaphoreType.DMA,
            pltpu.SemaphoreType.DMA,
            pltpu.SemaphoreType.DMA,
        ],
    )
    def gk(t1_h, t2_h, p0_h, p1_h, g1_h, g2_h, ib0, ib1, b1, b2,
           s1a, s1b, s2a, s2b):
        wid = lax.axis_index("s") * 2 + lax.axis_index("c")
        base = wid * PER_W
        pltpu.sync_copy(p0_h.at[pl.ds(base, PER_W)], ib0)
        pltpu.sync_copy(p1_h.at[pl.ds(base, PER_W)], ib1)
        s1 = (s1a, s1b)
        s2 = (s2a, s2b)

        def issue(c, slot):
            off = c * CHUNK
            pltpu.async_copy(t1_h.at[ib0.at[pl.ds(off, CHUNK)]],
                             b1.at[slot], s1[slot])
            pltpu.async_copy(t2_h.at[ib1.at[pl.ds(off, CHUNK)]],
                             b2.at[slot], s2[slot])

        issue(0, 0)
        issue(1, 1)

        def body(i, carry):
            c = i * 2
            for slot in range(2):
                cc = c + slot
                pltpu.make_async_copy(t1_h.at[pl.ds(0, CHUNK)],
                                      b1.at[slot], s1[slot]).wait()
                pltpu.make_async_copy(t2_h.at[pl.ds(0, CHUNK)],
                                      b2.at[slot], s2[slot]).wait()
                pltpu.sync_copy(b1.at[slot],
                                g1_h.at[pl.ds(base + cc * CHUNK, CHUNK)])
                pltpu.sync_copy(b2.at[slot],
                                g2_h.at[pl.ds(base + cc * CHUNK, CHUNK)])

                @pl.when(cc + 2 < NCHUNK)
                def _():
                    issue(cc + 2, slot)
            return carry

        lax.fori_loop(0, NCHUNK // 2, body, 0)

    return gk(t1, t2, p0, p1)


def _build_pairs(edge_coords):
    diff = edge_coords[:, None, :] - edge_coords[None, :, :]
    dist = jnp.sqrt(jnp.sum(diff * diff, axis=-1))
    _, idx = jax.lax.top_k(-dist, TOP_K)
    mask = jnp.zeros((E, E), dtype=bool).at[jnp.arange(E)[:, None], idx].set(True)
    mask = mask | mask.T
    p0, p1 = jnp.nonzero(mask, size=P, fill_value=E)
    return p0, p1


def kernel(edge_features, edge_coords, Wq, Wk, Wv, aw1, ab1, aw2, ab2, aw3,
           ab3, gw1, gb1, gw2, gb2, ow, ob, ln_g, ln_b):
    p0, p1 = _build_pairs(jax.lax.stop_gradient(edge_coords))

    q = edge_features @ Wq.T
    k = edge_features @ Wk.T
    v = edge_features @ Wv.T

    vh = v.reshape(E, HEADS, HEAD_DIM)
    g1 = _silu(jnp.einsum('ehd,od->eho', vh, gw1) + gb1)
    gate = jax.nn.sigmoid(jnp.einsum('eho,xo->ehx', g1, gw2)[..., 0] + gb2[0])

    def pack2(x):
        b = jax.lax.bitcast_convert_type(x.astype(jnp.bfloat16), jnp.uint16)
        w = (b[:, 1::2].astype(jnp.uint32) << 16) | b[:, 0::2].astype(jnp.uint32)
        return jax.lax.bitcast_convert_type(w, jnp.float32)

    t1 = jnp.zeros((E + 8, D1), jnp.float32).at[:E, :HIDDEN // 2].set(pack2(q))
    t2 = jnp.zeros((E + 8, D2), jnp.float32)
    t2 = (t2.at[:E, :HIDDEN // 2].set(pack2(k))
            .at[:E, HIDDEN // 2:HIDDEN].set(pack2(v)))
    g1, g2 = _sc_gather(t1, t2, p0.astype(jnp.int32), p1.astype(jnp.int32))

    vw = jax.lax.bitcast_convert_type(g2[:, HIDDEN // 2:HIDDEN], jnp.uint32)
    ve = jax.lax.bitcast_convert_type(vw << 16, jnp.float32)
    vo = jax.lax.bitcast_convert_type(vw & jnp.uint32(0xFFFF0000), jnp.float32)
    vg = jnp.stack([ve, vo], axis=-1).reshape(P, HIDDEN)

    gg = gate[p1]
    cd = edge_coords[p0] - edge_coords[p1]
    cd4 = jnp.pad(cd, ((0, 0), (0, 1)))

    perm = jnp.array([2 * i for i in range(HEAD_DIM // 2)]
                     + [2 * i + 1 for i in range(HEAD_DIM // 2)])
    aqt = aw1[:, :HEAD_DIM].T[perm]
    akt = aw1[:, HEAD_DIM:2 * HEAD_DIM].T[perm]
    art = aw1[:, 2 * HEAD_DIM:2 * HEAD_DIM + NUM_RADIAL].T
    ad = aw1[:, -1][None, :]
    cent = jnp.linspace(0.0, CUTOFF, NUM_RADIAL)[None, :]

    ex = _pair_scores(g1, g2, cd4, aqt, akt, art, ad, ab1[None, :],
                      aw2.T, ab2[None, :], aw3.T, cent, ab3[None, :])

    z = jax.ops.segment_sum(ex, p0, num_segments=E)
    attn = ex / (z[p0] + 1e-16)

    wv = (attn[:, :, None] * vg.reshape(P, HEADS, HEAD_DIM)).reshape(P, HIDDEN)
    agg = jax.ops.segment_sum(wv, p0, num_segments=E)

    w = jnp.sum(attn * gg, axis=-1, keepdims=True)
    cu = jax.ops.segment_sum(w * cd, p0, num_segments=E)
    updated_coords = edge_coords + cu / HEADS

    x = edge_features + agg @ ow.T + ob
    mu = jnp.mean(x, axis=-1, keepdims=True)
    var = jnp.mean((x - mu) ** 2, axis=-1, keepdims=True)
    normed = (x - mu) / jnp.sqrt(var + 1e-5) * ln_g + ln_b
    return normed, updated_coords


# speedup vs baseline: 1.1451x; 1.1451x over previous
import functools

import jax
import jax.numpy as jnp
from jax import lax
from jax.experimental import pallas as pl
from jax.experimental.pallas import tpu as pltpu
from jax.experimental.pallas import tpu_sc as plsc

E = 2048
HIDDEN = 128
HEADS = 8
HEAD_DIM = HIDDEN // HEADS
NUM_RADIAL = 64
CUTOFF = 10.0
TOP_K = 32
P = 2 * E * TOP_K

BLK = 2048


def _silu(x):
    return x * jax.nn.sigmoid(x)


def _unpack_pair(w):
    wi = jax.lax.bitcast_convert_type(w, jnp.uint32)
    lo = jax.lax.bitcast_convert_type(wi << 16, jnp.float32)
    hi = jax.lax.bitcast_convert_type(wi & jnp.uint32(0xFFFF0000), jnp.float32)
    return lo, hi


def _pair_mlp_body(g1_ref, g2_ref, cd_ref, aqt_ref, akt_ref, art_ref, ad_ref,
                   ab1_ref, aw2t_ref, ab2_ref, aw3t_ref, cent_ref, ab3_ref,
                   out_ref):
    qe, qo = _unpack_pair(g1_ref[:, :HIDDEN // 2])
    ke, ko = _unpack_pair(g2_ref[:, :HIDDEN // 2])
    cd = cd_ref[...]
    d2 = jnp.sum(cd * cd, axis=-1, keepdims=True)
    d = jnp.sqrt(d2 + 1e-12)
    gamma = (NUM_RADIAL / CUTOFF) ** 2
    cent = cent_ref[...]
    rf = jnp.exp(-gamma * (d - cent) ** 2)
    rfc = jnp.dot(rf, art_ref[...], preferred_element_type=jnp.float32)
    ab1 = ab1_ref[...]
    aw2t = aw2t_ref[...]
    ab2 = ab2_ref[...]
    aw3t = aw3t_ref[...]
    ad = ad_ref[...]
    HW = HEAD_DIM // 2
    for h in range(HEADS):
        qp = jnp.concatenate([qe[:, h * HW:(h + 1) * HW],
                              qo[:, h * HW:(h + 1) * HW]], axis=1)
        kp = jnp.concatenate([ke[:, h * HW:(h + 1) * HW],
                              ko[:, h * HW:(h + 1) * HW]], axis=1)
        dp = jnp.sum(qp * kp, axis=-1, keepdims=True)
        pre = (jnp.dot(qp, aqt_ref[...], preferred_element_type=jnp.float32)
               + jnp.dot(kp, akt_ref[...], preferred_element_type=jnp.float32)
               + rfc + dp * ad + ab1)
        h1 = _silu(pre)
        h2 = _silu(jnp.dot(h1, aw2t, preferred_element_type=jnp.float32) + ab2)
        s = jnp.dot(h2, aw3t[:, h:h + 1], preferred_element_type=jnp.float32)
        out_ref[:, h:h + 1] = jnp.exp(s + ab3_ref[0:1, h:h + 1])


def _pair_scores(g1, g2, cd4, aqt, akt, art, ad, ab1, aw2t, ab2, aw3t, cent,
                 ab3):
    nblk = P // BLK
    row = lambda i: (i, 0)
    fixed = lambda i: (0, 0)
    return pl.pallas_call(
        _pair_mlp_body,
        grid=(nblk,),
        in_specs=[
            pl.BlockSpec((BLK, D1), row),
            pl.BlockSpec((BLK, D2), row),
            pl.BlockSpec((BLK, 4), row),
            pl.BlockSpec((HEAD_DIM, HIDDEN), fixed),
            pl.BlockSpec((HEAD_DIM, HIDDEN), fixed),
            pl.BlockSpec((NUM_RADIAL, HIDDEN), fixed),
            pl.BlockSpec((1, HIDDEN), fixed),
            pl.BlockSpec((1, HIDDEN), fixed),
            pl.BlockSpec((HIDDEN, HIDDEN), fixed),
            pl.BlockSpec((1, HIDDEN), fixed),
            pl.BlockSpec((HIDDEN, HEADS), fixed),
            pl.BlockSpec((1, NUM_RADIAL), fixed),
            pl.BlockSpec((1, HEADS), fixed),
        ],
        out_specs=pl.BlockSpec((BLK, HEADS), row),
        out_shape=jax.ShapeDtypeStruct((P, HEADS), jnp.float32),
    )(g1, g2, cd4, aqt, akt, art, ad, ab1, aw2t, ab2, aw3t, cent, ab3)


D1 = 128
D2 = 128
NW = 32
CHUNK = 128
PER_W = P // NW
NCHUNK = PER_W // CHUNK


def _sc_gather(t1, t2, p0, p1):
    mesh = plsc.VectorSubcoreMesh(core_axis_name="c", subcore_axis_name="s")

    @functools.partial(
        pl.kernel,
        out_type=(jax.ShapeDtypeStruct((P, D1), jnp.float32),
                  jax.ShapeDtypeStruct((P, D2), jnp.float32)),
        mesh=mesh,
        scratch_types=[
            pltpu.VMEM((PER_W,), jnp.int32),
            pltpu.VMEM((PER_W,), jnp.int32),
            pltpu.VMEM((2, CHUNK, D1), jnp.float32),
            pltpu.VMEM((2, CHUNK, D2), jnp.float32),
            pltpu.SemaphoreType.DMA,
            pltpu.SemaphoreType.DMA,
            pltpu.SemaphoreType.DMA,
            pltpu.SemaphoreType.DMA,
        ],
    )
    def gk(t1_h, t2_h, p0_h, p1_h, g1_h, g2_h, ib0, ib1, b1, b2,
           s1a, s1b, s2a, s2b):
        wid = lax.axis_index("s") * 2 + lax.axis_index("c")
        base = wid * PER_W
        pltpu.sync_copy(p0_h.at[pl.ds(base, PER_W)], ib0)
        pltpu.sync_copy(p1_h.at[pl.ds(base, PER_W)], ib1)
        s1 = (s1a, s1b)
        s2 = (s2a, s2b)

        def issue(c, slot):
            off = c * CHUNK
            pltpu.async_copy(t1_h.at[ib0.at[pl.ds(off, CHUNK)]],
                             b1.at[slot], s1[slot])
            pltpu.async_copy(t2_h.at[ib1.at[pl.ds(off, CHUNK)]],
                             b2.at[slot], s2[slot])

        issue(0, 0)
        issue(1, 1)

        def body(i, carry):
            c = i * 2
            for slot in range(2):
                cc = c + slot
                pltpu.make_async_copy(t1_h.at[pl.ds(0, CHUNK)],
                                      b1.at[slot], s1[slot]).wait()
                pltpu.make_async_copy(t2_h.at[pl.ds(0, CHUNK)],
                                      b2.at[slot], s2[slot]).wait()
                pltpu.sync_copy(b1.at[slot],
                                g1_h.at[pl.ds(base + cc * CHUNK, CHUNK)])
                pltpu.sync_copy(b2.at[slot],
                                g2_h.at[pl.ds(base + cc * CHUNK, CHUNK)])

                @pl.when(cc + 2 < NCHUNK)
                def _():
                    issue(cc + 2, slot)
            return carry

        lax.fori_loop(0, NCHUNK // 2, body, 0)

    return gk(t1, t2, p0, p1)


def _build_pairs(edge_coords):
    diff = edge_coords[:, None, :] - edge_coords[None, :, :]
    dist = jnp.sqrt(jnp.sum(diff * diff, axis=-1))
    _, idx = jax.lax.top_k(-dist, TOP_K)
    mask = jnp.zeros((E, E), dtype=bool).at[jnp.arange(E)[:, None], idx].set(True)
    mask = mask | mask.T
    p0, p1 = jnp.nonzero(mask, size=P, fill_value=E)
    return p0, p1


def kernel(edge_features, edge_coords, Wq, Wk, Wv, aw1, ab1, aw2, ab2, aw3,
           ab3, gw1, gb1, gw2, gb2, ow, ob, ln_g, ln_b):
    p0, p1 = _build_pairs(jax.lax.stop_gradient(edge_coords))

    q = edge_features @ Wq.T
    k = edge_features @ Wk.T
    v = edge_features @ Wv.T

    vh = v.reshape(E, HEADS, HEAD_DIM)
    g1 = _silu(jnp.einsum('ehd,od->eho', vh, gw1) + gb1)
    gate = jax.nn.sigmoid(jnp.einsum('eho,xo->ehx', g1, gw2)[..., 0] + gb2[0])

    def pack2(x):
        b = jax.lax.bitcast_convert_type(x.astype(jnp.bfloat16), jnp.uint16)
        w = (b[:, 1::2].astype(jnp.uint32) << 16) | b[:, 0::2].astype(jnp.uint32)
        return jax.lax.bitcast_convert_type(w, jnp.float32)

    t1 = jnp.zeros((E + 8, D1), jnp.float32).at[:E, :HIDDEN // 2].set(pack2(q))
    t2 = jnp.zeros((E + 8, D2), jnp.float32)
    t2 = (t2.at[:E, :HIDDEN // 2].set(pack2(k))
            .at[:E, HIDDEN // 2:HIDDEN].set(pack2(v)))
    g1 = t1[p0]
    g2 = t2[p1]

    vw = jax.lax.bitcast_convert_type(g2[:, HIDDEN // 2:HIDDEN], jnp.uint32)
    ve = jax.lax.bitcast_convert_type(vw << 16, jnp.float32)
    vo = jax.lax.bitcast_convert_type(vw & jnp.uint32(0xFFFF0000), jnp.float32)
    vg = jnp.stack([ve, vo], axis=-1).reshape(P, HIDDEN)

    gg = gate[p1]
    cd = edge_coords[p0] - edge_coords[p1]
    cd4 = jnp.pad(cd, ((0, 0), (0, 1)))

    perm = jnp.array([2 * i for i in range(HEAD_DIM // 2)]
                     + [2 * i + 1 for i in range(HEAD_DIM // 2)])
    aqt = aw1[:, :HEAD_DIM].T[perm]
    akt = aw1[:, HEAD_DIM:2 * HEAD_DIM].T[perm]
    art = aw1[:, 2 * HEAD_DIM:2 * HEAD_DIM + NUM_RADIAL].T
    ad = aw1[:, -1][None, :]
    cent = jnp.linspace(0.0, CUTOFF, NUM_RADIAL)[None, :]

    ex = _pair_scores(g1, g2, cd4, aqt, akt, art, ad, ab1[None, :],
                      aw2.T, ab2[None, :], aw3.T, cent, ab3[None, :])

    z = jax.ops.segment_sum(ex, p0, num_segments=E)
    attn = ex / (z[p0] + 1e-16)

    wv = (attn[:, :, None] * vg.reshape(P, HEADS, HEAD_DIM)).reshape(P, HIDDEN)
    agg = jax.ops.segment_sum(wv, p0, num_segments=E)

    w = jnp.sum(attn * gg, axis=-1, keepdims=True)
    cu = jax.ops.segment_sum(w * cd, p0, num_segments=E)
    updated_coords = edge_coords + cu / HEADS

    x = edge_features + agg @ ow.T + ob
    mu = jnp.mean(x, axis=-1, keepdims=True)
    var = jnp.mean((x - mu) ** 2, axis=-1, keepdims=True)
    normed = (x - mu) / jnp.sqrt(var + 1e-5) * ln_g + ln_b
    return normed, updated_coords


# speedup vs baseline: 1.1959x; 1.0444x over previous
import functools

import jax
import jax.numpy as jnp
from jax import lax
from jax.experimental import pallas as pl
from jax.experimental.pallas import tpu as pltpu
from jax.experimental.pallas import tpu_sc as plsc

E = 2048
HIDDEN = 128
HEADS = 8
HEAD_DIM = HIDDEN // HEADS
NUM_RADIAL = 64
CUTOFF = 10.0
TOP_K = 32
P = 2 * E * TOP_K

BLK = 2048


def _silu(x):
    return x * jax.nn.sigmoid(x)


def _unpack_pair(w):
    wi = jax.lax.bitcast_convert_type(w, jnp.uint32)
    lo = jax.lax.bitcast_convert_type(wi << 16, jnp.float32)
    hi = jax.lax.bitcast_convert_type(wi & jnp.uint32(0xFFFF0000), jnp.float32)
    return lo, hi


def _pair_mlp_body(g1_ref, g2_ref, cd_ref, aqt_ref, akt_ref, art_ref, ad_ref,
                   ab1_ref, aw2t_ref, ab2_ref, aw3t_ref, cent_ref, ab3_ref,
                   out_ref):
    qe, qo = _unpack_pair(g1_ref[:, :HIDDEN // 2])
    ke, ko = _unpack_pair(g2_ref[:, :HIDDEN // 2])
    cd = cd_ref[...]
    d2 = jnp.sum(cd * cd, axis=-1, keepdims=True)
    d = jnp.sqrt(d2 + 1e-12)
    gamma = (NUM_RADIAL / CUTOFF) ** 2
    cent = cent_ref[...]
    rf = jnp.exp(-gamma * (d - cent) ** 2)
    rfc = jnp.dot(rf, art_ref[...], preferred_element_type=jnp.float32)
    ab1 = ab1_ref[...]
    aw2t = aw2t_ref[...]
    ab2 = ab2_ref[...]
    aw3t = aw3t_ref[...]
    ad = ad_ref[...]
    HW = HEAD_DIM // 2
    for h in range(HEADS):
        qp = jnp.concatenate([qe[:, h * HW:(h + 1) * HW],
                              qo[:, h * HW:(h + 1) * HW]], axis=1)
        kp = jnp.concatenate([ke[:, h * HW:(h + 1) * HW],
                              ko[:, h * HW:(h + 1) * HW]], axis=1)
        dp = jnp.sum(qp * kp, axis=-1, keepdims=True)
        pre = (jnp.dot(qp, aqt_ref[...], preferred_element_type=jnp.float32)
               + jnp.dot(kp, akt_ref[...], preferred_element_type=jnp.float32)
               + rfc + dp * ad + ab1)
        h1 = _silu(pre)
        h2 = _silu(jnp.dot(h1, aw2t, preferred_element_type=jnp.float32) + ab2)
        s = jnp.dot(h2, aw3t[:, h:h + 1], preferred_element_type=jnp.float32)
        out_ref[:, h:h + 1] = jnp.exp(s + ab3_ref[0:1, h:h + 1])


def _pair_scores(g1, g2, cd4, aqt, akt, art, ad, ab1, aw2t, ab2, aw3t, cent,
                 ab3):
    nblk = P // BLK
    row = lambda i: (i, 0)
    fixed = lambda i: (0, 0)
    return pl.pallas_call(
        _pair_mlp_body,
        grid=(nblk,),
        in_specs=[
            pl.BlockSpec((BLK, D1), row),
            pl.BlockSpec((BLK, D2), row),
            pl.BlockSpec((BLK, 4), row),
            pl.BlockSpec((HEAD_DIM, HIDDEN), fixed),
            pl.BlockSpec((HEAD_DIM, HIDDEN), fixed),
            pl.BlockSpec((NUM_RADIAL, HIDDEN), fixed),
            pl.BlockSpec((1, HIDDEN), fixed),
            pl.BlockSpec((1, HIDDEN), fixed),
            pl.BlockSpec((HIDDEN, HIDDEN), fixed),
            pl.BlockSpec((1, HIDDEN), fixed),
            pl.BlockSpec((HIDDEN, HEADS), fixed),
            pl.BlockSpec((1, NUM_RADIAL), fixed),
            pl.BlockSpec((1, HEADS), fixed),
        ],
        out_specs=pl.BlockSpec((BLK, HEADS), row),
        out_shape=jax.ShapeDtypeStruct((P, HEADS), jnp.float32),
    )(g1, g2, cd4, aqt, akt, art, ad, ab1, aw2t, ab2, aw3t, cent, ab3)


D1 = 128
D2 = 128
NW = 32
CHUNK = 128
PER_W = P // NW
NCHUNK = PER_W // CHUNK


def _sc_gather(t1, t2, p0, p1):
    mesh = plsc.VectorSubcoreMesh(core_axis_name="c", subcore_axis_name="s")

    @functools.partial(
        pl.kernel,
        out_type=(jax.ShapeDtypeStruct((P, D1), jnp.float32),
                  jax.ShapeDtypeStruct((P, D2), jnp.float32)),
        mesh=mesh,
        scratch_types=[
            pltpu.VMEM((PER_W,), jnp.int32),
            pltpu.VMEM((PER_W,), jnp.int32),
            pltpu.VMEM((2, CHUNK, D1), jnp.float32),
            pltpu.VMEM((2, CHUNK, D2), jnp.float32),
            pltpu.SemaphoreType.DMA,
            pltpu.SemaphoreType.DMA,
            pltpu.SemaphoreType.DMA,
            pltpu.SemaphoreType.DMA,
        ],
    )
    def gk(t1_h, t2_h, p0_h, p1_h, g1_h, g2_h, ib0, ib1, b1, b2,
           s1a, s1b, s2a, s2b):
        wid = lax.axis_index("s") * 2 + lax.axis_index("c")
        base = wid * PER_W
        pltpu.sync_copy(p0_h.at[pl.ds(base, PER_W)], ib0)
        pltpu.sync_copy(p1_h.at[pl.ds(base, PER_W)], ib1)
        s1 = (s1a, s1b)
        s2 = (s2a, s2b)

        def issue(c, slot):
            off = c * CHUNK
            pltpu.async_copy(t1_h.at[ib0.at[pl.ds(off, CHUNK)]],
                             b1.at[slot], s1[slot])
            pltpu.async_copy(t2_h.at[ib1.at[pl.ds(off, CHUNK)]],
                             b2.at[slot], s2[slot])

        issue(0, 0)
        issue(1, 1)

        def body(i, carry):
            c = i * 2
            for slot in range(2):
                cc = c + slot
                pltpu.make_async_copy(t1_h.at[pl.ds(0, CHUNK)],
                                      b1.at[slot], s1[slot]).wait()
                pltpu.make_async_copy(t2_h.at[pl.ds(0, CHUNK)],
                                      b2.at[slot], s2[slot]).wait()
                pltpu.sync_copy(b1.at[slot],
                                g1_h.at[pl.ds(base + cc * CHUNK, CHUNK)])
                pltpu.sync_copy(b2.at[slot],
                                g2_h.at[pl.ds(base + cc * CHUNK, CHUNK)])

                @pl.when(cc + 2 < NCHUNK)
                def _():
                    issue(cc + 2, slot)
            return carry

        lax.fori_loop(0, NCHUNK // 2, body, 0)

    return gk(t1, t2, p0, p1)


AGGW = 136


def _agg_body(ex_ref, g2_ref, gg_ref, cd_ref, zinv_ref, p0_ref, agg_ref):
    @pl.when(pl.program_id(0) == 0)
    def _():
        agg_ref[...] = jnp.zeros_like(agg_ref)

    p0row = p0_ref[...].reshape(1, BLK)
    rows = jax.lax.broadcasted_iota(jnp.int32, (E, 1), 0)
    onehot_t = (rows == p0row).astype(jnp.bfloat16)

    ex = ex_ref[...]
    zinv = zinv_ref[...]
    zh = zinv.astype(jnp.bfloat16)
    zl = (zinv - zh.astype(jnp.float32)).astype(jnp.bfloat16)
    dn = (((0,), (0,)), ((), ()))
    zb = (jax.lax.dot_general(onehot_t, zh, dn,
                              preferred_element_type=jnp.float32)
          + jax.lax.dot_general(onehot_t, zl, dn,
                                preferred_element_type=jnp.float32))
    attn = ex * zb
    ve, vo = _unpack_pair(g2_ref[:, HIDDEN // 2:HIDDEN])
    attnx = jnp.broadcast_to(attn[:, :, None],
                             (BLK, HEADS, HEAD_DIM // 2)).reshape(BLK, 64)
    w = jnp.sum(attn * gg_ref[...], axis=-1, keepdims=True)
    payload = jnp.concatenate(
        [ve * attnx, vo * attnx, w * cd_ref[...],
         jnp.zeros((BLK, AGGW - 132), jnp.float32)], axis=1)
    ph = payload.astype(jnp.bfloat16)
    pl_ = (payload - ph.astype(jnp.float32)).astype(jnp.bfloat16)
    dn2 = (((1,), (0,)), ((), ()))
    part = (jax.lax.dot_general(onehot_t, ph, dn2,
                                preferred_element_type=jnp.float32)
            + jax.lax.dot_general(onehot_t, pl_, dn2,
                                  preferred_element_type=jnp.float32))
    agg_ref[...] += part


def _agg_scatter(ex, g2, gg, cd4, zinv, p0_3d):
    nblk = P // BLK
    row = lambda i: (i, 0)
    fixed = lambda i: (0, 0)
    return pl.pallas_call(
        _agg_body,
        grid=(nblk,),
        in_specs=[
            pl.BlockSpec((BLK, HEADS), row),
            pl.BlockSpec((BLK, D2), row),
            pl.BlockSpec((BLK, HEADS), row),
            pl.BlockSpec((BLK, 4), row),
            pl.BlockSpec((E, HEADS), fixed),
            pl.BlockSpec((1, 1, BLK), lambda i: (i, 0, 0)),
        ],
        out_specs=pl.BlockSpec((E, AGGW), fixed),
        out_shape=jax.ShapeDtypeStruct((E, AGGW), jnp.float32),
    )(ex, g2, gg, cd4, zinv, p0_3d)


def _build_pairs(edge_coords):
    diff = edge_coords[:, None, :] - edge_coords[None, :, :]
    dist = jnp.sqrt(jnp.sum(diff * diff, axis=-1))
    _, idx = jax.lax.top_k(-dist, TOP_K)
    mask = jnp.zeros((E, E), dtype=bool).at[jnp.arange(E)[:, None], idx].set(True)
    mask = mask | mask.T
    p0, p1 = jnp.nonzero(mask, size=P, fill_value=E)
    return p0, p1


def kernel(edge_features, edge_coords, Wq, Wk, Wv, aw1, ab1, aw2, ab2, aw3,
           ab3, gw1, gb1, gw2, gb2, ow, ob, ln_g, ln_b):
    p0, p1 = _build_pairs(jax.lax.stop_gradient(edge_coords))

    q = edge_features @ Wq.T
    k = edge_features @ Wk.T
    v = edge_features @ Wv.T

    vh = v.reshape(E, HEADS, HEAD_DIM)
    g1 = _silu(jnp.einsum('ehd,od->eho', vh, gw1) + gb1)
    gate = jax.nn.sigmoid(jnp.einsum('eho,xo->ehx', g1, gw2)[..., 0] + gb2[0])

    def pack2(x):
        b = jax.lax.bitcast_convert_type(x.astype(jnp.bfloat16), jnp.uint16)
        w = (b[:, 1::2].astype(jnp.uint32) << 16) | b[:, 0::2].astype(jnp.uint32)
        return jax.lax.bitcast_convert_type(w, jnp.float32)

    t1 = jnp.zeros((E + 8, D1), jnp.float32).at[:E, :HIDDEN // 2].set(pack2(q))
    t2 = jnp.zeros((E + 8, D2), jnp.float32)
    t2 = (t2.at[:E, :HIDDEN // 2].set(pack2(k))
            .at[:E, HIDDEN // 2:HIDDEN].set(pack2(v)))
    g1 = t1[p0]
    g2 = t2[p1]

    gg = gate[p1]
    cd = edge_coords[p0] - edge_coords[p1]
    cd4 = jnp.pad(cd, ((0, 0), (0, 1)))

    perm = jnp.array([2 * i for i in range(HEAD_DIM // 2)]
                     + [2 * i + 1 for i in range(HEAD_DIM // 2)])
    aqt = aw1[:, :HEAD_DIM].T[perm]
    akt = aw1[:, HEAD_DIM:2 * HEAD_DIM].T[perm]
    art = aw1[:, 2 * HEAD_DIM:2 * HEAD_DIM + NUM_RADIAL].T
    ad = aw1[:, -1][None, :]
    cent = jnp.linspace(0.0, CUTOFF, NUM_RADIAL)[None, :]

    ex = _pair_scores(g1, g2, cd4, aqt, akt, art, ad, ab1[None, :],
                      aw2.T, ab2[None, :], aw3.T, cent, ab3[None, :])

    z = jax.ops.segment_sum(ex, p0, num_segments=E)
    zinv = 1.0 / (z + 1e-16)
    p0_3d = p0.astype(jnp.int32).reshape(P // BLK, 1, BLK)
    acc = _agg_scatter(ex, g2, gg, cd4, zinv, p0_3d)
    agg = jnp.stack([acc[:, :64], acc[:, 64:128]], axis=-1).reshape(E, HIDDEN)
    cu = acc[:, 128:131]
    updated_coords = edge_coords + cu / HEADS

    x = edge_features + agg @ ow.T + ob
    mu = jnp.mean(x, axis=-1, keepdims=True)
    var = jnp.mean((x - mu) ** 2, axis=-1, keepdims=True)
    normed = (x - mu) / jnp.sqrt(var + 1e-5) * ln_g + ln_b
    return normed, updated_coords


# speedup vs baseline: 1.3777x; 1.1520x over previous
import functools

import jax
import jax.numpy as jnp
from jax import lax
from jax.experimental import pallas as pl
from jax.experimental.pallas import tpu as pltpu
from jax.experimental.pallas import tpu_sc as plsc

E = 2048
HIDDEN = 128
HEADS = 8
HEAD_DIM = HIDDEN // HEADS
NUM_RADIAL = 64
CUTOFF = 10.0
TOP_K = 32
P = 2 * E * TOP_K

BLK = 2048


def _silu(x):
    return x * jax.nn.sigmoid(x)


def _unpack_pair(w):
    wi = jax.lax.bitcast_convert_type(w, jnp.uint32)
    lo = jax.lax.bitcast_convert_type(wi << 16, jnp.float32)
    hi = jax.lax.bitcast_convert_type(wi & jnp.uint32(0xFFFF0000), jnp.float32)
    return lo, hi


def _pair_mlp_body(g1_ref, g2_ref, cd_ref, aqt_ref, akt_ref, art_ref, ad_ref,
                   ab1_ref, aw2t_ref, ab2_ref, aw3t_ref, cent_ref, ab3_ref,
                   out_ref):
    qe, qo = _unpack_pair(g1_ref[:, :HIDDEN // 2])
    ke, ko = _unpack_pair(g2_ref[:, :HIDDEN // 2])
    cd = cd_ref[...]
    d2 = jnp.sum(cd * cd, axis=-1, keepdims=True)
    d = jnp.sqrt(d2 + 1e-12)
    gamma = (NUM_RADIAL / CUTOFF) ** 2
    cent = cent_ref[...]
    rf = jnp.exp(-gamma * (d - cent) ** 2)
    rfc = jnp.dot(rf, art_ref[...], preferred_element_type=jnp.float32)
    ab1 = ab1_ref[...]
    aw2t = aw2t_ref[...]
    ab2 = ab2_ref[...]
    aw3t = aw3t_ref[...]
    ad = ad_ref[...]
    HW = HEAD_DIM // 2
    for h in range(HEADS):
        qp = jnp.concatenate([qe[:, h * HW:(h + 1) * HW],
                              qo[:, h * HW:(h + 1) * HW]], axis=1)
        kp = jnp.concatenate([ke[:, h * HW:(h + 1) * HW],
                              ko[:, h * HW:(h + 1) * HW]], axis=1)
        dp = jnp.sum(qp * kp, axis=-1, keepdims=True)
        pre = (jnp.dot(qp, aqt_ref[...], preferred_element_type=jnp.float32)
               + jnp.dot(kp, akt_ref[...], preferred_element_type=jnp.float32)
               + rfc + dp * ad + ab1)
        h1 = _silu(pre)
        h2 = _silu(jnp.dot(h1, aw2t, preferred_element_type=jnp.float32) + ab2)
        s = jnp.dot(h2, aw3t[:, h:h + 1], preferred_element_type=jnp.float32)
        out_ref[:, h:h + 1] = jnp.exp(s + ab3_ref[0:1, h:h + 1])


def _pair_scores(g1, g2, cd4, aqt, akt, art, ad, ab1, aw2t, ab2, aw3t, cent,
                 ab3):
    nblk = P // BLK
    row = lambda i: (i, 0)
    fixed = lambda i: (0, 0)
    return pl.pallas_call(
        _pair_mlp_body,
        grid=(nblk,),
        in_specs=[
            pl.BlockSpec((BLK, D1), row),
            pl.BlockSpec((BLK, D2), row),
            pl.BlockSpec((BLK, 4), row),
            pl.BlockSpec((HEAD_DIM, HIDDEN), fixed),
            pl.BlockSpec((HEAD_DIM, HIDDEN), fixed),
            pl.BlockSpec((NUM_RADIAL, HIDDEN), fixed),
            pl.BlockSpec((1, HIDDEN), fixed),
            pl.BlockSpec((1, HIDDEN), fixed),
            pl.BlockSpec((HIDDEN, HIDDEN), fixed),
            pl.BlockSpec((1, HIDDEN), fixed),
            pl.BlockSpec((HIDDEN, HEADS), fixed),
            pl.BlockSpec((1, NUM_RADIAL), fixed),
            pl.BlockSpec((1, HEADS), fixed),
        ],
        out_specs=pl.BlockSpec((BLK, HEADS), row),
        out_shape=jax.ShapeDtypeStruct((P, HEADS), jnp.float32),
    )(g1, g2, cd4, aqt, akt, art, ad, ab1, aw2t, ab2, aw3t, cent, ab3)


D1 = 128
D2 = 128
NW = 32
CHUNK = 128
PER_W = P // NW
NCHUNK = PER_W // CHUNK


def _sc_gather(t1, t2, p0, p1):
    mesh = plsc.VectorSubcoreMesh(core_axis_name="c", subcore_axis_name="s")

    @functools.partial(
        pl.kernel,
        out_type=(jax.ShapeDtypeStruct((P, D1), jnp.float32),
                  jax.ShapeDtypeStruct((P, D2), jnp.float32)),
        mesh=mesh,
        scratch_types=[
            pltpu.VMEM((PER_W,), jnp.int32),
            pltpu.VMEM((PER_W,), jnp.int32),
            pltpu.VMEM((2, CHUNK, D1), jnp.float32),
            pltpu.VMEM((2, CHUNK, D2), jnp.float32),
            pltpu.SemaphoreType.DMA,
            pltpu.SemaphoreType.DMA,
            pltpu.SemaphoreType.DMA,
            pltpu.SemaphoreType.DMA,
        ],
    )
    def gk(t1_h, t2_h, p0_h, p1_h, g1_h, g2_h, ib0, ib1, b1, b2,
           s1a, s1b, s2a, s2b):
        wid = lax.axis_index("s") * 2 + lax.axis_index("c")
        base = wid * PER_W
        pltpu.sync_copy(p0_h.at[pl.ds(base, PER_W)], ib0)
        pltpu.sync_copy(p1_h.at[pl.ds(base, PER_W)], ib1)
        s1 = (s1a, s1b)
        s2 = (s2a, s2b)

        def issue(c, slot):
            off = c * CHUNK
            pltpu.async_copy(t1_h.at[ib0.at[pl.ds(off, CHUNK)]],
                             b1.at[slot], s1[slot])
            pltpu.async_copy(t2_h.at[ib1.at[pl.ds(off, CHUNK)]],
                             b2.at[slot], s2[slot])

        issue(0, 0)
        issue(1, 1)

        def body(i, carry):
            c = i * 2
            for slot in range(2):
                cc = c + slot
                pltpu.make_async_copy(t1_h.at[pl.ds(0, CHUNK)],
                                      b1.at[slot], s1[slot]).wait()
                pltpu.make_async_copy(t2_h.at[pl.ds(0, CHUNK)],
                                      b2.at[slot], s2[slot]).wait()
                pltpu.sync_copy(b1.at[slot],
                                g1_h.at[pl.ds(base + cc * CHUNK, CHUNK)])
                pltpu.sync_copy(b2.at[slot],
                                g2_h.at[pl.ds(base + cc * CHUNK, CHUNK)])

                @pl.when(cc + 2 < NCHUNK)
                def _():
                    issue(cc + 2, slot)
            return carry

        lax.fori_loop(0, NCHUNK // 2, body, 0)

    return gk(t1, t2, p0, p1)


AGGW = 136


def _agg_body(ex_ref, g2_ref, gg_ref, cd_ref, zinv_ref, p0_ref, agg_ref):
    @pl.when(pl.program_id(0) == 0)
    def _():
        agg_ref[...] = jnp.zeros_like(agg_ref)

    p0row = p0_ref[...].reshape(1, BLK)
    rows = jax.lax.broadcasted_iota(jnp.int32, (E, 1), 0)
    onehot_t = (rows == p0row).astype(jnp.bfloat16)

    ex = ex_ref[...]
    zinv = zinv_ref[...]
    zh = zinv.astype(jnp.bfloat16)
    zl = (zinv - zh.astype(jnp.float32)).astype(jnp.bfloat16)
    dn = (((0,), (0,)), ((), ()))
    zb = (jax.lax.dot_general(onehot_t, zh, dn,
                              preferred_element_type=jnp.float32)
          + jax.lax.dot_general(onehot_t, zl, dn,
                                preferred_element_type=jnp.float32))
    attn = ex * zb
    ve, vo = _unpack_pair(g2_ref[:, HIDDEN // 2:HIDDEN])
    attnx = jnp.broadcast_to(attn[:, :, None],
                             (BLK, HEADS, HEAD_DIM // 2)).reshape(BLK, 64)
    w = jnp.sum(attn * gg_ref[...], axis=-1, keepdims=True)
    payload = jnp.concatenate(
        [ve * attnx, vo * attnx, w * cd_ref[...],
         jnp.zeros((BLK, AGGW - 132), jnp.float32)], axis=1)
    ph = payload.astype(jnp.bfloat16)
    pl_ = (payload - ph.astype(jnp.float32)).astype(jnp.bfloat16)
    dn2 = (((1,), (0,)), ((), ()))
    part = (jax.lax.dot_general(onehot_t, ph, dn2,
                                preferred_element_type=jnp.float32)
            + jax.lax.dot_general(onehot_t, pl_, dn2,
                                  preferred_element_type=jnp.float32))
    agg_ref[...] += part


def _agg_scatter(ex, g2, gg, cd4, zinv, p0_3d):
    nblk = P // BLK
    row = lambda i: (i, 0)
    fixed = lambda i: (0, 0)
    return pl.pallas_call(
        _agg_body,
        grid=(nblk,),
        in_specs=[
            pl.BlockSpec((BLK, HEADS), row),
            pl.BlockSpec((BLK, D2), row),
            pl.BlockSpec((BLK, HEADS), row),
            pl.BlockSpec((BLK, 4), row),
            pl.BlockSpec((E, HEADS), fixed),
            pl.BlockSpec((1, 1, BLK), lambda i: (i, 0, 0)),
        ],
        out_specs=pl.BlockSpec((E, AGGW), fixed),
        out_shape=jax.ShapeDtypeStruct((E, AGGW), jnp.float32),
    )(ex, g2, gg, cd4, zinv, p0_3d)


def _build_pairs(edge_coords):
    diff = edge_coords[:, None, :] - edge_coords[None, :, :]
    d2 = jnp.sum(diff * diff, axis=-1)
    bits = jax.lax.bitcast_convert_type(d2, jnp.int32)

    def step(c, _):
        lo, hi = c
        mid = lo + ((hi - lo) >> 1)
        cnt = jnp.sum((bits <= mid[:, None]).astype(jnp.int32), axis=1)
        ge = cnt >= TOP_K
        return (jnp.where(ge, lo, mid + 1), jnp.where(ge, mid, hi)), 0.0

    init = (jnp.zeros((E,), jnp.int32),
            jnp.full((E,), jnp.int32(0x7F7FFFFF)))
    (lo, hi), _ = jax.lax.scan(step, init, None, length=31)
    tb = hi
    mask = bits <= jnp.maximum(tb[:, None], tb[None, :])
    p0, p1 = jnp.nonzero(mask, size=P, fill_value=E)
    return p0, p1


def kernel(edge_features, edge_coords, Wq, Wk, Wv, aw1, ab1, aw2, ab2, aw3,
           ab3, gw1, gb1, gw2, gb2, ow, ob, ln_g, ln_b):
    p0, p1 = _build_pairs(jax.lax.stop_gradient(edge_coords))

    q = edge_features @ Wq.T
    k = edge_features @ Wk.T
    v = edge_features @ Wv.T

    vh = v.reshape(E, HEADS, HEAD_DIM)
    g1 = _silu(jnp.einsum('ehd,od->eho', vh, gw1) + gb1)
    gate = jax.nn.sigmoid(jnp.einsum('eho,xo->ehx', g1, gw2)[..., 0] + gb2[0])

    def pack2(x):
        b = jax.lax.bitcast_convert_type(x.astype(jnp.bfloat16), jnp.uint16)
        w = (b[:, 1::2].astype(jnp.uint32) << 16) | b[:, 0::2].astype(jnp.uint32)
        return jax.lax.bitcast_convert_type(w, jnp.float32)

    t1 = jnp.zeros((E + 8, D1), jnp.float32).at[:E, :HIDDEN // 2].set(pack2(q))
    t2 = jnp.zeros((E + 8, D2), jnp.float32)
    t2 = (t2.at[:E, :HIDDEN // 2].set(pack2(k))
            .at[:E, HIDDEN // 2:HIDDEN].set(pack2(v)))
    g1 = t1[p0]
    g2 = t2[p1]

    gg = gate[p1]
    cd = edge_coords[p0] - edge_coords[p1]
    cd4 = jnp.pad(cd, ((0, 0), (0, 1)))

    perm = jnp.array([2 * i for i in range(HEAD_DIM // 2)]
                     + [2 * i + 1 for i in range(HEAD_DIM // 2)])
    aqt = aw1[:, :HEAD_DIM].T[perm]
    akt = aw1[:, HEAD_DIM:2 * HEAD_DIM].T[perm]
    art = aw1[:, 2 * HEAD_DIM:2 * HEAD_DIM + NUM_RADIAL].T
    ad = aw1[:, -1][None, :]
    cent = jnp.linspace(0.0, CUTOFF, NUM_RADIAL)[None, :]

    ex = _pair_scores(g1, g2, cd4, aqt, akt, art, ad, ab1[None, :],
                      aw2.T, ab2[None, :], aw3.T, cent, ab3[None, :])

    z = jax.ops.segment_sum(ex, p0, num_segments=E)
    zinv = 1.0 / (z + 1e-16)
    p0_3d = p0.astype(jnp.int32).reshape(P // BLK, 1, BLK)
    acc = _agg_scatter(ex, g2, gg, cd4, zinv, p0_3d)
    agg = jnp.stack([acc[:, :64], acc[:, 64:128]], axis=-1).reshape(E, HIDDEN)
    cu = acc[:, 128:131]
    updated_coords = edge_coords + cu / HEADS

    x = edge_features + agg @ ow.T + ob
    mu = jnp.mean(x, axis=-1, keepdims=True)
    var = jnp.mean((x - mu) ** 2, axis=-1, keepdims=True)
    normed = (x - mu) / jnp.sqrt(var + 1e-5) * ln_g + ln_b
    return normed, updated_coords
